# baseline (speedup 1.0000x reference)
import jax
import jax.numpy as jnp
from jax.experimental import pallas as pl
from jax.experimental.pallas import tpu as pltpu
from jax.experimental.pallas import tpu_sc as plsc


def _sc_add(x2, w):
    R, D = x2.shape
    L, _ = w.shape
    B = R // L
    BR, BC = 16, 1024
    NL = L // BR

    mesh = plsc.VectorSubcoreMesh(
        core_axis_name="core", subcore_axis_name="subcore"
    )

    @pl.kernel(out_type=jax.ShapeDtypeStruct((R, D), x2.dtype), mesh=mesh)
    def run(x_hbm, w_hbm, o_hbm):
        def body(x_vmem, w_vmem, o_vmem):
            @pl.loop(0, BR)
            def _(r):
                @pl.loop(0, BC, step=16)
                def _(c):
                    rs, cs = pl.ds(r, 1), pl.ds(c, 16)
                    o_vmem.at[rs, cs][...] = (
                        x_vmem.at[rs, cs][...] + w_vmem.at[rs, cs][...]
                    )

        pltpu.emit_pipeline(
            body,
            grid=(NL, B),
            in_specs=[
                pl.BlockSpec((BR, BC), lambda l, b: (b * NL + l, 0)),
                pl.BlockSpec((BR, BC), lambda l, b: (l, 0)),
            ],
            out_specs=[pl.BlockSpec((BR, BC), lambda l, b: (b * NL + l, 0))],
            core_axis_name=("core", "subcore"),
            dimension_semantics=(pltpu.PARALLEL, pltpu.ARBITRARY),
        )(x_hbm, w_hbm, o_hbm)

    return run(x2, w)


def kernel(x, weight):
    B, L, D = x.shape
    out2 = _sc_add(x.reshape(B * L, D), weight[:L])
    return out2.reshape(B, L, D)


# speedup vs baseline: 1.1778x; 1.1778x over previous
import jax
import jax.numpy as jnp
from jax.experimental import pallas as pl
from jax.experimental.pallas import tpu as pltpu
from jax.experimental.pallas import tpu_sc as plsc


def _sc_add(x2, w):
    R, D = x2.shape
    L, _ = w.shape
    B = R // L
    BR, BC = 4, 1024
    NL = L // BR

    mesh = plsc.VectorSubcoreMesh(
        core_axis_name="core", subcore_axis_name="subcore"
    )

    @pl.kernel(out_type=jax.ShapeDtypeStruct((R, D), x2.dtype), mesh=mesh)
    def run(x_hbm, w_hbm, o_hbm):
        def body(w_vmem, *xo):
            xs, os_ = xo[:B], xo[B:]

            @pl.loop(0, BR)
            def _(r):
                rs = pl.ds(r, 1)
                for c in range(0, BC, 16):
                    cs = pl.ds(c, 16)
                    wv = w_vmem.at[rs, cs][...]
                    for b in range(B):
                        os_[b].at[rs, cs][...] = xs[b].at[rs, cs][...] + wv

        def x_spec(b):
            return pl.BlockSpec((BR, BC), lambda l, b=b: (b * NL + l, 0))

        pltpu.emit_pipeline(
            body,
            grid=(NL,),
            in_specs=[pl.BlockSpec((BR, BC), lambda l: (l, 0))]
            + [x_spec(b) for b in range(B)],
            out_specs=[x_spec(b) for b in range(B)],
            core_axis_name=("core", "subcore"),
            dimension_semantics=(pltpu.PARALLEL,),
        )(w_hbm, *([x_hbm] * B), *([o_hbm] * B))

    return run(x2, w)


def kernel(x, weight):
    B, L, D = x.shape
    out2 = _sc_add(x.reshape(B * L, D), weight[:L])
    return out2.reshape(B, L, D)


# speedup vs baseline: 2.2401x; 1.9019x over previous
import functools

import jax
import jax.numpy as jnp
from jax import lax
from jax.experimental import pallas as pl
from jax.experimental.pallas import tpu as pltpu
from jax.experimental.pallas import tpu_sc as plsc

_NC, _NS = 2, 16
_NW = _NC * _NS


def _sc_add(x2, w):
    R, D = x2.shape
    L, _ = w.shape
    rows_per_w = R // _NW
    CH = 16
    NCH = rows_per_w // CH

    mesh = plsc.VectorSubcoreMesh(core_axis_name="c", subcore_axis_name="s")

    @functools.partial(
        pl.kernel,
        mesh=mesh,
        out_type=jax.ShapeDtypeStruct((R, D), jnp.float32),
        scratch_types=[
            pltpu.VMEM((CH, D), jnp.float32),
            pltpu.VMEM((CH, D), jnp.float32),
            pltpu.VMEM((CH, D), jnp.float32),
            pltpu.VMEM((CH, D), jnp.float32),
            pltpu.VMEM((CH, D), jnp.float32),
            pltpu.VMEM((CH, D), jnp.float32),
            pltpu.SemaphoreType.DMA,
            pltpu.SemaphoreType.DMA,
            pltpu.SemaphoreType.DMA,
            pltpu.SemaphoreType.DMA,
            pltpu.SemaphoreType.DMA,
            pltpu.SemaphoreType.DMA,
        ],
    )
    def run(x_hbm, w_hbm, o_hbm, xA, xB, wA, wB, oA, oB,
            sxA, sxB, swA, swB, soA, soB):
        wid = lax.axis_index("s") * _NC + lax.axis_index("c")
        base = wid * rows_per_w
        wbase = base % L

        def in_copies(k, xbuf, wbuf, sx, sw):
            return (
                pltpu.make_async_copy(
                    x_hbm.at[pl.ds(base + k * CH, CH)], xbuf, sx
                ),
                pltpu.make_async_copy(
                    w_hbm.at[pl.ds(wbase + k * CH, CH)], wbuf, sw
                ),
            )

        def out_copy(k, obuf, so):
            return pltpu.make_async_copy(
                obuf, o_hbm.at[pl.ds(base + k * CH, CH)], so
            )

        for cp in in_copies(0, xA, wA, sxA, swA):
            cp.start()
        for cp in in_copies(1, xB, wB, sxB, swB):
            cp.start()

        def step(k, xbuf, wbuf, obuf, sx, sw, so):
            for cp in in_copies(k, xbuf, wbuf, sx, sw):
                cp.wait()

            @pl.when(k >= 2)
            def _():
                out_copy(k - 2, obuf, so).wait()

            @pl.loop(0, CH)
            def _(r):
                rs = pl.ds(r, 1)
                for c in range(0, D, 16):
                    cs = pl.ds(c, 16)
                    obuf.at[rs, cs][...] = (
                        xbuf.at[rs, cs][...] + wbuf.at[rs, cs][...]
                    )

            out_copy(k, obuf, so).start()

            @pl.when(k + 2 < NCH)
            def _():
                for cp in in_copies(k + 2, xbuf, wbuf, sx, sw):
                    cp.start()

        @pl.loop(0, NCH, step=2)
        def _(k):
            step(k, xA, wA, oA, sxA, swA, soA)
            step(k + 1, xB, wB, oB, sxB, swB, soB)

        out_copy(NCH - 2, oA, soA).wait()
        out_copy(NCH - 1, oB, soB).wait()

    return run(x2, w)


def kernel(x, weight):
    B, L, D = x.shape
    out2 = _sc_add(x.reshape(B * L, D), weight[:L])
    return out2.reshape(B, L, D)
